# baseline (speedup 1.0000x reference)
import jax
import jax.numpy as jnp
from jax import lax
from jax.experimental import pallas as pl
from jax.experimental.pallas import tpu as pltpu
from jax.experimental.pallas import tpu_sc as plsc

BATCH = 16384
RANK_K = 32
H1 = 256
H2 = 64
LW = 128
NUM_WORKERS = 32
B_PER_W = BATCH // NUM_WORKERS


def _gather_body(u_tab, v_tab, i_hbm, j_hbm, u_out, v_out,
                 idx, qidx, rows, sem):
    wid = lax.axis_index("s") * 2 + lax.axis_index("c")
    base = wid * B_PER_W
    for (src, tab, dst) in ((i_hbm, u_tab, u_out), (j_hbm, v_tab, v_out)):
        pltpu.sync_copy(src.at[pl.ds(base, B_PER_W)], idx)
        for s in range(B_PER_W // 16):
            qidx[pl.ds(s * 16, 16)] = lax.shift_right_logical(
                idx[pl.ds(s * 16, 16)], 2)
        pltpu.async_copy(tab.at[qidx], rows, sem).wait()
        pltpu.sync_copy(rows, dst.at[pl.ds(base, B_PER_W)])


def _make_gather(n_u_rows, n_v_rows):
    mesh = plsc.VectorSubcoreMesh(core_axis_name="c", subcore_axis_name="s")
    return pl.kernel(
        _gather_body,
        out_type=(
            jax.ShapeDtypeStruct((BATCH, LW), jnp.float32),
            jax.ShapeDtypeStruct((BATCH, LW), jnp.float32),
        ),
        mesh=mesh,
        scratch_types=[
            pltpu.VMEM((B_PER_W,), jnp.int32),
            pltpu.VMEM((B_PER_W,), jnp.int32),
            pltpu.VMEM((B_PER_W, LW), jnp.float32),
            pltpu.SemaphoreType.DMA,
        ],
        compiler_params=pltpu.CompilerParams(use_tc_tiling_on_sc=False),
    )


def _mlp_body(u_ref, v_ref, i_ref, j_ref, w1u_ref, w1v_ref, w2_ref, wl_ref,
              out_ref):
    bb = u_ref.shape[0]
    lane_grp = lax.shift_right_logical(
        lax.broadcasted_iota(jnp.int32, (bb, LW), 1), 5)
    mu = (lane_grp == (i_ref[...] & 3)).astype(jnp.float32)
    mv = (lane_grp == (j_ref[...] & 3)).astype(jnp.float32)
    h = lax.dot_general(u_ref[...] * mu, w1u_ref[...],
                        (((1,), (1,)), ((), ())),
                        preferred_element_type=jnp.float32)
    h = h + lax.dot_general(v_ref[...] * mv, w1v_ref[...],
                            (((1,), (1,)), ((), ())),
                            preferred_element_type=jnp.float32)
    h = 0.5 * h * (1.0 + lax.erf(h * 0.7071067811865476))
    y = lax.dot_general(h, w2_ref[...], (((1,), (1,)), ((), ())),
                        preferred_element_type=jnp.float32)
    out_ref[...] = jnp.sum(y * wl_ref[...], axis=1)


def _make_mlp(bb):
    return pl.pallas_call(
        _mlp_body,
        grid=(BATCH // bb,),
        in_specs=[
            pl.BlockSpec((bb, LW), lambda b: (b, 0)),
            pl.BlockSpec((bb, LW), lambda b: (b, 0)),
            pl.BlockSpec((bb, 1), lambda b: (b, 0)),
            pl.BlockSpec((bb, 1), lambda b: (b, 0)),
            pl.BlockSpec((H1, LW), lambda b: (0, 0)),
            pl.BlockSpec((H1, LW), lambda b: (0, 0)),
            pl.BlockSpec((H2, H1), lambda b: (0, 0)),
            pl.BlockSpec((1, H2), lambda b: (0, 0)),
        ],
        out_specs=pl.BlockSpec((bb,), lambda b: (b,)),
        out_shape=jax.ShapeDtypeStruct((BATCH,), jnp.float32),
    )


def kernel(i, j, U, V, W1, W2, Wl):
    i = i.astype(jnp.int32)
    j = j.astype(jnp.int32)
    u_tab = U.reshape(U.shape[0] * RANK_K // LW, LW)
    v_tab = V.reshape(V.shape[0] * RANK_K // LW, LW)
    u128, v128 = _make_gather(u_tab.shape[0], v_tab.shape[0])(u_tab, v_tab, i, j)
    w1ux = jnp.tile(W1[:, :RANK_K], (1, LW // RANK_K))
    w1vx = jnp.tile(W1[:, RANK_K:], (1, LW // RANK_K))
    return _make_mlp(2048)(u128, v128, i.reshape(BATCH, 1), j.reshape(BATCH, 1),
                           w1ux, w1vx, W2, Wl)


# speedup vs baseline: 1.0225x; 1.0225x over previous
import jax
import jax.numpy as jnp
from jax import lax
from jax.experimental import pallas as pl
from jax.experimental.pallas import tpu as pltpu
from jax.experimental.pallas import tpu_sc as plsc

BATCH = 16384
RANK_K = 32
H1 = 256
H2 = 64
LW = 128
NUM_WORKERS = 32
B_PER_W = BATCH // NUM_WORKERS


def _gather_body(u_tab, v_tab, i_hbm, j_hbm, u_out, v_out,
                 idx_i, idx_j, rows, sem):
    wid = lax.axis_index("s") * 2 + lax.axis_index("c")
    base = wid * B_PER_W
    pltpu.sync_copy(i_hbm.at[pl.ds(base, B_PER_W)], idx_i)
    pltpu.sync_copy(j_hbm.at[pl.ds(base, B_PER_W)], idx_j)
    pltpu.async_copy(u_tab.at[idx_i], rows, sem).wait()
    pltpu.sync_copy(rows, u_out.at[pl.ds(base, B_PER_W)])
    pltpu.async_copy(v_tab.at[idx_j], rows, sem).wait()
    pltpu.sync_copy(rows, v_out.at[pl.ds(base, B_PER_W)])


def _make_gather():
    mesh = plsc.VectorSubcoreMesh(core_axis_name="c", subcore_axis_name="s")
    return pl.kernel(
        _gather_body,
        out_type=(
            jax.ShapeDtypeStruct((BATCH, LW), jnp.float32),
            jax.ShapeDtypeStruct((BATCH, LW), jnp.float32),
        ),
        mesh=mesh,
        scratch_types=[
            pltpu.VMEM((B_PER_W,), jnp.int32),
            pltpu.VMEM((B_PER_W,), jnp.int32),
            pltpu.VMEM((B_PER_W, LW), jnp.float32),
            pltpu.SemaphoreType.DMA,
        ],
    )


def _mlp_body(u_ref, v_ref, w1u_ref, w1v_ref, w2_ref, wl_ref, out_ref):
    h = lax.dot_general(u_ref[...], w1u_ref[...], (((1,), (1,)), ((), ())),
                        preferred_element_type=jnp.float32)
    h = h + lax.dot_general(v_ref[...], w1v_ref[...], (((1,), (1,)), ((), ())),
                            preferred_element_type=jnp.float32)
    h = 0.5 * h * (1.0 + lax.erf(h * 0.7071067811865476))
    y = lax.dot_general(h, w2_ref[...], (((1,), (1,)), ((), ())),
                        preferred_element_type=jnp.float32)
    out_ref[...] = jnp.sum(y * wl_ref[...], axis=1)


def _make_mlp(bb):
    return pl.pallas_call(
        _mlp_body,
        grid=(BATCH // bb,),
        in_specs=[
            pl.BlockSpec((bb, LW), lambda b: (b, 0)),
            pl.BlockSpec((bb, LW), lambda b: (b, 0)),
            pl.BlockSpec((H1, LW), lambda b: (0, 0)),
            pl.BlockSpec((H1, LW), lambda b: (0, 0)),
            pl.BlockSpec((H2, H1), lambda b: (0, 0)),
            pl.BlockSpec((1, H2), lambda b: (0, 0)),
        ],
        out_specs=pl.BlockSpec((bb,), lambda b: (b,)),
        out_shape=jax.ShapeDtypeStruct((BATCH,), jnp.float32),
    )


def kernel(i, j, U, V, W1, W2, Wl):
    i = i.astype(jnp.int32)
    j = j.astype(jnp.int32)
    u_pad = jnp.pad(U, ((0, 0), (0, LW - RANK_K)))
    v_pad = jnp.pad(V, ((0, 0), (0, LW - RANK_K)))
    u128, v128 = _make_gather()(u_pad, v_pad, i, j)
    w1u = jnp.pad(W1[:, :RANK_K], ((0, 0), (0, LW - RANK_K)))
    w1v = jnp.pad(W1[:, RANK_K:], ((0, 0), (0, LW - RANK_K)))
    return _make_mlp(2048)(u128, v128, w1u, w1v, W2, Wl)
